# baseline (speedup 1.0000x reference)
import math

import jax
import jax.numpy as jnp
from jax import lax
from jax.experimental import pallas as pl
from jax.experimental.pallas import tpu as pltpu
from jax.experimental.pallas import tpu_sc as plsc

_DIM = 3
_N_LEVELS = 16
_N_FEATS = 2
_LOG2_HASHMAP = 19
_BASE_RES = 16
_FINEST_RES = 1024
_N = 524288

_PRIMES = (1, 2654435761, 805459861)
_b = math.exp((math.log(_FINEST_RES) - math.log(_BASE_RES)) / (_N_LEVELS - 1))
_RES = [math.floor(_BASE_RES * (_b ** i)) for i in range(_N_LEVELS)]
_MSIZE = [min(r ** _DIM, 2 ** _LOG2_HASHMAP) for r in _RES]

_NC = 2
_NS = 16
_LANES = 16
_NW = _NC * _NS
_PPW = _N // _NW
_C = 256
_NCHUNK = _PPW // _C
_G = 8 * _C

_NB = [((m + 1023) // 1024) * 8 for m in _MSIZE]
_NCH = [nb // 8 for nb in _NB]
_OFF4 = []
_acc = 0
for _nb in _NB:
    _OFF4.append(_acc)
    _acc += _nb * 32
_TOT4 = _acc

_OB = _N_LEVELS * _N_FEATS // 8
_RB = _N // 128
_N_SP_LEVELS = 6
_SP4 = _OFF4[_N_SP_LEVELS]
_HB4 = _TOT4 - _SP4
_OFFL = [_OFF4[l] if l < _N_SP_LEVELS else _OFF4[l] - _SP4
         for l in range(_N_LEVELS)]


def _mod_const(h, m):
    if m & (m - 1) == 0:
        return (h & jnp.uint32(m - 1)).astype(jnp.int32)
    c = jnp.float32(256.0 / m)
    hf = (h >> jnp.uint32(8)).astype(jnp.int32).astype(jnp.float32)
    q = (hf * c).astype(jnp.int32).astype(jnp.uint32)
    r = h - q * jnp.uint32(m)
    r = jnp.where(r >= jnp.uint32(0x80000000), r + jnp.uint32(m), r)
    r = jnp.where(r >= jnp.uint32(m), r - jnp.uint32(m), r)
    return r.astype(jnp.int32)


def _make_kernel():
    mesh = plsc.VectorSubcoreMesh(core_axis_name="c", subcore_axis_name="s")

    def repack_level(l, tv, dst_ref, sid, lanes, inbuf, rpbuf,
                     sem_in, sem_out):
        nch = _NCH[l]
        off4 = _OFFL[l]
        lr = lanes >> 2
        lc2 = (lanes * 2) & 7

        def in_copy(k, slot):
            q = sid + k * _NS
            return pltpu.make_async_copy(
                tv.at[pl.ds(q * 8, 8)], inbuf.at[slot], sem_in[slot])

        def out_copy(k, slot):
            q = sid + k * _NS
            return pltpu.make_async_copy(
                rpbuf.at[slot], dst_ref.at[pl.ds(off4 + q * 256, 256)],
                sem_out[slot])

        cnt = (nch - sid + _NS - 1) // _NS

        @pl.when(cnt > 0)
        def _():
            in_copy(0, 0).start()

        @pl.when(cnt > 1)
        def _():
            in_copy(1, 1).start()

        def process(k, slot):
            in_copy(k, slot).wait()

            @pl.when(k >= 2)
            def _():
                out_copy(k - 2, slot).wait()

            def ileave(t, c2):
                blk = t >> 4
                f = (t >> 3) & 1
                jv = t & 7
                v = inbuf[slot, blk, f, pl.ds(jv * _LANES, _LANES)]
                rows = blk * 32 + jv * 4 + lr
                cols = lc2 + f
                plsc.store_scatter(rpbuf.at[slot], [rows, cols], v)
                return c2

            lax.fori_loop(0, 128, ileave, 0)
            out_copy(k, slot).start()

            @pl.when(k + 2 < cnt)
            def _():
                in_copy(k + 2, slot).start()

        def body(k2, carry):
            for slot in (0, 1):
                k = k2 * 2 + slot

                @pl.when(k < cnt)
                def _():
                    process(k, slot)

            return carry

        lax.fori_loop(0, (cnt + 1) // 2, body, 0)

        for slot in (0, 1):
            k_s = ((cnt - 1 - slot) // 2) * 2 + slot

            @pl.when((k_s >= 0) & (k_s < cnt))
            def _():
                out_copy(k_s, slot).wait()

    def compute_level(l, slot, xslot, xbuf, idxbuf, subbuf, wbuf):
        res = float(_RES[l])
        m = _MSIZE[l]
        off4 = _OFFL[l]

        def body(i, carry):
            s = pl.ds(i * _LANES, _LANES)
            h_lo, h_hi, w_lo, w_hi = [], [], [], []
            for d in range(_DIM):
                xs = xbuf[xslot, d, s] * jnp.float32(res)
                xi = xs.astype(jnp.int32)
                xf = xs - xi.astype(jnp.float32)
                xu = xi.astype(jnp.uint32)
                p = jnp.uint32(_PRIMES[d])
                if d == 0:
                    h_lo.append(xu)
                    h_hi.append(xu + jnp.uint32(1))
                else:
                    h_lo.append(xu * p)
                    h_hi.append((xu + jnp.uint32(1)) * p)
                w_lo.append(jnp.float32(1.0) - xf)
                w_hi.append(xf)
            for cn in range(8):
                h = ((h_hi[0] if cn & 1 else h_lo[0])
                     ^ (h_hi[1] if cn & 2 else h_lo[1])
                     ^ (h_hi[2] if cn & 4 else h_lo[2]))
                hid = _mod_const(h, m)
                idxbuf[slot, pl.ds(cn * _C + i * _LANES, _LANES)] = (
                    off4 + (hid >> 2))
                subbuf[slot, cn, s] = (hid & 3) * 2
                w = ((w_hi[0] if cn & 1 else w_lo[0])
                     * (w_hi[1] if cn & 2 else w_lo[1])
                     * (w_hi[2] if cn & 4 else w_lo[2]))
                wbuf[slot, cn, s] = w
            return carry

        lax.fori_loop(0, _C // _LANES, body, 0)

    def gather_copy(table, slot, idxbuf, rows, sem):
        src = table.at[idxbuf.at[slot]]
        dst = rows.at[slot]
        return pltpu.make_async_copy(src, dst, sem)

    def interp_level(l, slot, rows, subbuf, wbuf, obuf, lanes):
        ones = jnp.full((_LANES,), 1, jnp.int32)
        cb = l >> 2
        cc = (2 * l) & 7

        def body(i, carry):
            s = pl.ds(i * _LANES, _LANES)
            pts = i * _LANES + lanes
            p0 = i * _LANES
            rb = p0 >> 7
            ro = p0 & 127
            a0 = jnp.zeros((_LANES,), jnp.float32)
            a1 = jnp.zeros((_LANES,), jnp.float32)
            for cn in range(8):
                w = wbuf[slot, cn, s]
                sub2 = subbuf[slot, cn, s]
                rowids = cn * _C + pts
                f0 = plsc.load_gather(rows.at[slot], [rowids, sub2])
                f1 = plsc.load_gather(rows.at[slot], [rowids, sub2 + ones])
                a0 = a0 + w * f0
                a1 = a1 + w * f1
            obuf[cb, rb, cc, pl.ds(ro, _LANES)] = a0
            obuf[cb, rb, cc + 1, pl.ds(ro, _LANES)] = a1
            return carry

        lax.fori_loop(0, _C // _LANES, body, 0)

    def body(xT, t00, t01, t02, t03, t04, t05, t06, t07, t08, t09, t10, t11,
             t12, t13, t14, t15, outk, packed, xbuf, idxbuf, subbuf, rows,
             wbuf, obuf, inbuf, rpbuf, spbuf, sem0, sem1, sin0, sin1,
             sout0, sout1, sem_x):
        tvs = [t00, t01, t02, t03, t04, t05, t06, t07, t08, t09, t10, t11,
               t12, t13, t14, t15]
        sems = [sem0, sem1]
        sem_in = [sin0, sin1]
        sem_out = [sout0, sout1]
        cid = lax.axis_index("c")
        sid = lax.axis_index("s")
        wid = sid * _NC + cid
        wbase = wid * _PPW
        lanes = lax.iota(jnp.int32, _LANES)

        for l in range(_N_LEVELS):
            dst_ref = spbuf if l < _N_SP_LEVELS else packed
            repack_level(l, tvs[l], dst_ref, sid, lanes, inbuf, rpbuf,
                         sem_in, sem_out)
        plsc.subcore_barrier()

        order = list(range(_N_LEVELS))
        srcs = [spbuf if l < _N_SP_LEVELS else packed
                for l in range(_N_LEVELS)]

        def xload(ch, xslot):
            base = wbase + ch * _C
            return pltpu.make_async_copy(
                xT.at[:, pl.ds(base, _C)], xbuf.at[xslot], sem_x)

        xload(0, 0).start()
        xload(0, 0).wait()
        compute_level(order[0], 0, 0, xbuf, idxbuf, subbuf, wbuf)
        gather_copy(srcs[order[0]], 0, idxbuf, rows, sems[0]).start()

        def chunk_body(ch, carry):
            xslot = ch & 1
            rb_ch = (wbase + ch * _C) >> 7

            @pl.when(ch + 1 < _NCHUNK)
            def _():
                xload(ch + 1, 1 - xslot).start()

            for pos in range(1, _N_LEVELS):
                slot = pos & 1
                lv, pv = order[pos], order[pos - 1]
                compute_level(lv, slot, xslot, xbuf, idxbuf, subbuf, wbuf)
                gather_copy(srcs[lv], slot, idxbuf, rows, sems[slot]).start()
                gather_copy(srcs[pv], 1 - slot, idxbuf, rows,
                            sems[1 - slot]).wait()
                interp_level(pv, 1 - slot, rows, subbuf, wbuf, obuf, lanes)

            @pl.when(ch + 1 < _NCHUNK)
            def _():
                xload(ch + 1, 1 - xslot).wait()
                compute_level(order[0], 0, 1 - xslot, xbuf, idxbuf, subbuf,
                              wbuf)
                gather_copy(srcs[order[0]], 0, idxbuf, rows, sems[0]).start()

            last = order[_N_LEVELS - 1]
            gather_copy(srcs[last], 1, idxbuf, rows, sems[1]).wait()
            interp_level(last, 1, rows, subbuf, wbuf, obuf, lanes)
            pltpu.sync_copy(obuf, outk.at[:, pl.ds(rb_ch, _C // 128)])
            return carry

        lax.fori_loop(0, _NCHUNK, chunk_body, 0)

    return pl.kernel(
        body,
        mesh=mesh,
        compiler_params=pltpu.CompilerParams(
            needs_layout_passes=False, use_tc_tiling_on_sc=False),
        out_type=(
            jax.ShapeDtypeStruct((_OB, _RB, 8, 128), jnp.float32),
            jax.ShapeDtypeStruct((_HB4, 8), jnp.float32),
        ),
        scratch_types=[
            pltpu.VMEM((2, _DIM, _C), jnp.float32),
            pltpu.VMEM((2, _G), jnp.int32),
            pltpu.VMEM((2, 8, _C), jnp.int32),
            pltpu.VMEM((2, _G, 8), jnp.float32),
            pltpu.VMEM((2, 8, _C), jnp.float32),
            pltpu.VMEM((_OB, _C // 128, 8, 128), jnp.float32),
            pltpu.VMEM((2, 8, 2, 128), jnp.float32),
            pltpu.VMEM((2, 256, 8), jnp.float32),
            pltpu.VMEM_SHARED((_SP4, 8), jnp.float32),
            pltpu.SemaphoreType.DMA,
            pltpu.SemaphoreType.DMA,
            pltpu.SemaphoreType.DMA,
            pltpu.SemaphoreType.DMA,
            pltpu.SemaphoreType.DMA,
            pltpu.SemaphoreType.DMA,
            pltpu.SemaphoreType.DMA,
        ],
    )


_sc_kernel = _make_kernel()


@jax.jit
def kernel(x, table_00, table_01, table_02, table_03, table_04, table_05,
           table_06, table_07, table_08, table_09, table_10, table_11,
           table_12, table_13, table_14, table_15):
    xT = jnp.transpose(x)
    tabs = [table_00, table_01, table_02, table_03, table_04, table_05,
            table_06, table_07, table_08, table_09, table_10, table_11,
            table_12, table_13, table_14, table_15]
    tvs = []
    for i, t in enumerate(tabs):
        rows128 = _NB[i] * 128
        if rows128 != _MSIZE[i]:
            t = jnp.pad(t, ((0, rows128 - _MSIZE[i]), (0, 0)))
        tvs.append(jnp.transpose(t.reshape(_NB[i], 128, _N_FEATS), (0, 2, 1)))
    outk, _ = _sc_kernel(xT, *tvs)
    return jnp.transpose(outk, (1, 3, 0, 2)).reshape(_N, _N_LEVELS * _N_FEATS)


# speedup vs baseline: 1.0037x; 1.0037x over previous
import math

import jax
import jax.numpy as jnp
from jax import lax
from jax.experimental import pallas as pl
from jax.experimental.pallas import tpu as pltpu
from jax.experimental.pallas import tpu_sc as plsc

_DIM = 3
_N_LEVELS = 16
_N_FEATS = 2
_LOG2_HASHMAP = 19
_BASE_RES = 16
_FINEST_RES = 1024
_N = 524288

_PRIMES = (1, 2654435761, 805459861)
_b = math.exp((math.log(_FINEST_RES) - math.log(_BASE_RES)) / (_N_LEVELS - 1))
_RES = [math.floor(_BASE_RES * (_b ** i)) for i in range(_N_LEVELS)]
_MSIZE = [min(r ** _DIM, 2 ** _LOG2_HASHMAP) for r in _RES]

_NC = 2
_NS = 16
_LANES = 16
_NW = _NC * _NS
_PPW = _N // _NW
_C = 256
_NCHUNK = _PPW // _C
_G = 8 * _C

_NB = [((m + 1023) // 1024) * 8 for m in _MSIZE]
_NCH = [nb // 8 for nb in _NB]
_OFF4 = []
_acc = 0
for _nb in _NB:
    _OFF4.append(_acc)
    _acc += _nb * 32
_TOT4 = _acc

_OB = _N_LEVELS * _N_FEATS // 8
_RB = _N // 128
_N_SP_LEVELS = 6
_SP4 = _OFF4[_N_SP_LEVELS]
_HB4 = _TOT4 - _SP4
_OFFL = [_OFF4[l] if l < _N_SP_LEVELS else _OFF4[l] - _SP4
         for l in range(_N_LEVELS)]


def _mod_const(h, m):
    if m & (m - 1) == 0:
        return (h & jnp.uint32(m - 1)).astype(jnp.int32)
    c = jnp.float32(256.0 / m)
    hf = (h >> jnp.uint32(8)).astype(jnp.int32).astype(jnp.float32)
    q = (hf * c).astype(jnp.int32).astype(jnp.uint32)
    r = h - q * jnp.uint32(m)
    r = jnp.where(r >= jnp.uint32(0x80000000), r + jnp.uint32(m), r)
    r = jnp.where(r >= jnp.uint32(m), r - jnp.uint32(m), r)
    return r.astype(jnp.int32)


def _make_kernel():
    mesh = plsc.VectorSubcoreMesh(core_axis_name="c", subcore_axis_name="s")

    def repack_level(l, tv, dst_ref, sid, lanes, inbuf, rpbuf,
                     sem_in, sem_out):
        nch = _NCH[l]
        off4 = _OFFL[l]
        lr = lanes >> 2
        lc2 = (lanes * 2) & 7

        def in_copy(k, slot):
            q = sid + k * _NS
            return pltpu.make_async_copy(
                tv.at[pl.ds(q * 8, 8)], inbuf.at[slot], sem_in[slot])

        def out_copy(k, slot):
            q = sid + k * _NS
            return pltpu.make_async_copy(
                rpbuf.at[slot], dst_ref.at[pl.ds(off4 + q * 256, 256)],
                sem_out[slot])

        cnt = (nch - sid + _NS - 1) // _NS

        @pl.when(cnt > 0)
        def _():
            in_copy(0, 0).start()

        @pl.when(cnt > 1)
        def _():
            in_copy(1, 1).start()

        def process(k, slot):
            in_copy(k, slot).wait()

            @pl.when(k >= 2)
            def _():
                out_copy(k - 2, slot).wait()

            def ileave(t, c2):
                blk = t >> 4
                f = (t >> 3) & 1
                jv = t & 7
                v = inbuf[slot, blk, f, pl.ds(jv * _LANES, _LANES)]
                rows = blk * 32 + jv * 4 + lr
                cols = lc2 + f
                plsc.store_scatter(rpbuf.at[slot], [rows, cols], v)
                return c2

            lax.fori_loop(0, 128, ileave, 0)
            out_copy(k, slot).start()

            @pl.when(k + 2 < cnt)
            def _():
                in_copy(k + 2, slot).start()

        def body(k2, carry):
            for slot in (0, 1):
                k = k2 * 2 + slot

                @pl.when(k < cnt)
                def _():
                    process(k, slot)

            return carry

        lax.fori_loop(0, (cnt + 1) // 2, body, 0)

        for slot in (0, 1):
            k_s = ((cnt - 1 - slot) // 2) * 2 + slot

            @pl.when((k_s >= 0) & (k_s < cnt))
            def _():
                out_copy(k_s, slot).wait()

    def compute_level(l, slot, xslot, xbuf, idxbuf, subbuf, wbuf):
        res = float(_RES[l])
        m = _MSIZE[l]
        off4 = _OFFL[l]

        def body(i, carry):
            s = pl.ds(i * _LANES, _LANES)
            h_lo, h_hi, w_lo, w_hi = [], [], [], []
            for d in range(_DIM):
                xs = xbuf[xslot, d, s] * jnp.float32(res)
                xi = xs.astype(jnp.int32)
                xf = xs - xi.astype(jnp.float32)
                xu = xi.astype(jnp.uint32)
                p = jnp.uint32(_PRIMES[d])
                if d == 0:
                    h_lo.append(xu)
                    h_hi.append(xu + jnp.uint32(1))
                else:
                    h_lo.append(xu * p)
                    h_hi.append((xu + jnp.uint32(1)) * p)
                w_lo.append(jnp.float32(1.0) - xf)
                w_hi.append(xf)
            for cn in range(8):
                h = ((h_hi[0] if cn & 1 else h_lo[0])
                     ^ (h_hi[1] if cn & 2 else h_lo[1])
                     ^ (h_hi[2] if cn & 4 else h_lo[2]))
                hid = _mod_const(h, m)
                idxbuf[slot, pl.ds(cn * _C + i * _LANES, _LANES)] = (
                    off4 + (hid >> 2))
                subbuf[slot, cn, s] = (hid & 3) * 2
                w = ((w_hi[0] if cn & 1 else w_lo[0])
                     * (w_hi[1] if cn & 2 else w_lo[1])
                     * (w_hi[2] if cn & 4 else w_lo[2]))
                wbuf[slot, cn, s] = w
            return carry

        lax.fori_loop(0, _C // _LANES, body, 0)

    def gather_copy(table, slot, idxbuf, rows, sem):
        src = table.at[idxbuf.at[slot]]
        dst = rows.at[slot]
        return pltpu.make_async_copy(src, dst, sem)

    def interp_level(l, slot, oslot, rows, subbuf, wbuf, obuf, lanes):
        ones = jnp.full((_LANES,), 1, jnp.int32)
        cb = l >> 2
        cc = (2 * l) & 7

        def body(i, carry):
            s = pl.ds(i * _LANES, _LANES)
            pts = i * _LANES + lanes
            p0 = i * _LANES
            rb = p0 >> 7
            ro = p0 & 127
            a0 = jnp.zeros((_LANES,), jnp.float32)
            a1 = jnp.zeros((_LANES,), jnp.float32)
            for cn in range(8):
                w = wbuf[slot, cn, s]
                sub2 = subbuf[slot, cn, s]
                rowids = cn * _C + pts
                f0 = plsc.load_gather(rows.at[slot], [rowids, sub2])
                f1 = plsc.load_gather(rows.at[slot], [rowids, sub2 + ones])
                a0 = a0 + w * f0
                a1 = a1 + w * f1
            obuf[oslot, cb, rb, cc, pl.ds(ro, _LANES)] = a0
            obuf[oslot, cb, rb, cc + 1, pl.ds(ro, _LANES)] = a1
            return carry

        lax.fori_loop(0, _C // _LANES, body, 0)

    def body(xT, t00, t01, t02, t03, t04, t05, t06, t07, t08, t09, t10, t11,
             t12, t13, t14, t15, outk, packed, xbuf, idxbuf, subbuf, rows,
             wbuf, obuf, inbuf, rpbuf, spbuf, sem0, sem1, sin0, sin1,
             sout0, sout1, sem_x, sem_o):
        tvs = [t00, t01, t02, t03, t04, t05, t06, t07, t08, t09, t10, t11,
               t12, t13, t14, t15]
        sems = [sem0, sem1]
        sem_in = [sin0, sin1]
        sem_out = [sout0, sout1]
        cid = lax.axis_index("c")
        sid = lax.axis_index("s")
        wid = sid * _NC + cid
        wbase = wid * _PPW
        lanes = lax.iota(jnp.int32, _LANES)

        for l in range(_N_LEVELS):
            dst_ref = spbuf if l < _N_SP_LEVELS else packed
            repack_level(l, tvs[l], dst_ref, sid, lanes, inbuf, rpbuf,
                         sem_in, sem_out)
        plsc.subcore_barrier()

        order = list(range(_N_LEVELS))
        srcs = [spbuf if l < _N_SP_LEVELS else packed
                for l in range(_N_LEVELS)]

        def xload(ch, xslot):
            base = wbase + ch * _C
            return pltpu.make_async_copy(
                xT.at[:, pl.ds(base, _C)], xbuf.at[xslot], sem_x)

        xload(0, 0).start()
        xload(0, 0).wait()
        compute_level(order[0], 0, 0, xbuf, idxbuf, subbuf, wbuf)
        gather_copy(srcs[order[0]], 0, idxbuf, rows, sems[0]).start()

        def ostore(ch, oslot):
            rb_ch = (wbase + ch * _C) >> 7
            return pltpu.make_async_copy(
                obuf.at[oslot], outk.at[:, pl.ds(rb_ch, _C // 128)], sem_o)

        def chunk_body(ch, carry):
            xslot = ch & 1
            oslot = ch & 1

            @pl.when(ch + 1 < _NCHUNK)
            def _():
                xload(ch + 1, 1 - xslot).start()

            @pl.when(ch >= 2)
            def _():
                ostore(ch - 2, oslot).wait()

            for pos in range(1, _N_LEVELS):
                slot = pos & 1
                lv, pv = order[pos], order[pos - 1]
                compute_level(lv, slot, xslot, xbuf, idxbuf, subbuf, wbuf)
                gather_copy(srcs[lv], slot, idxbuf, rows, sems[slot]).start()
                gather_copy(srcs[pv], 1 - slot, idxbuf, rows,
                            sems[1 - slot]).wait()
                interp_level(pv, 1 - slot, oslot, rows, subbuf, wbuf, obuf,
                             lanes)

            @pl.when(ch + 1 < _NCHUNK)
            def _():
                xload(ch + 1, 1 - xslot).wait()
                compute_level(order[0], 0, 1 - xslot, xbuf, idxbuf, subbuf,
                              wbuf)
                gather_copy(srcs[order[0]], 0, idxbuf, rows, sems[0]).start()

            last = order[_N_LEVELS - 1]
            gather_copy(srcs[last], 1, idxbuf, rows, sems[1]).wait()
            interp_level(last, 1, oslot, rows, subbuf, wbuf, obuf, lanes)
            ostore(ch, oslot).start()
            return carry

        lax.fori_loop(0, _NCHUNK, chunk_body, 0)

        for oslot in (0, 1):
            k_s = ((_NCHUNK - 1 - oslot) // 2) * 2 + oslot
            if 0 <= k_s < _NCHUNK:
                ostore(k_s, oslot).wait()

    return pl.kernel(
        body,
        mesh=mesh,
        compiler_params=pltpu.CompilerParams(
            needs_layout_passes=False, use_tc_tiling_on_sc=False),
        out_type=(
            jax.ShapeDtypeStruct((_OB, _RB, 8, 128), jnp.float32),
            jax.ShapeDtypeStruct((_HB4, 8), jnp.float32),
        ),
        scratch_types=[
            pltpu.VMEM((2, _DIM, _C), jnp.float32),
            pltpu.VMEM((2, _G), jnp.int32),
            pltpu.VMEM((2, 8, _C), jnp.int32),
            pltpu.VMEM((2, _G, 8), jnp.float32),
            pltpu.VMEM((2, 8, _C), jnp.float32),
            pltpu.VMEM((2, _OB, _C // 128, 8, 128), jnp.float32),
            pltpu.VMEM((2, 8, 2, 128), jnp.float32),
            pltpu.VMEM((2, 256, 8), jnp.float32),
            pltpu.VMEM_SHARED((_SP4, 8), jnp.float32),
            pltpu.SemaphoreType.DMA,
            pltpu.SemaphoreType.DMA,
            pltpu.SemaphoreType.DMA,
            pltpu.SemaphoreType.DMA,
            pltpu.SemaphoreType.DMA,
            pltpu.SemaphoreType.DMA,
            pltpu.SemaphoreType.DMA,
            pltpu.SemaphoreType.DMA,
        ],
    )


_sc_kernel = _make_kernel()


@jax.jit
def kernel(x, table_00, table_01, table_02, table_03, table_04, table_05,
           table_06, table_07, table_08, table_09, table_10, table_11,
           table_12, table_13, table_14, table_15):
    xT = jnp.transpose(x)
    tabs = [table_00, table_01, table_02, table_03, table_04, table_05,
            table_06, table_07, table_08, table_09, table_10, table_11,
            table_12, table_13, table_14, table_15]
    tvs = []
    for i, t in enumerate(tabs):
        rows128 = _NB[i] * 128
        if rows128 != _MSIZE[i]:
            t = jnp.pad(t, ((0, rows128 - _MSIZE[i]), (0, 0)))
        tvs.append(jnp.transpose(t.reshape(_NB[i], 128, _N_FEATS), (0, 2, 1)))
    outk, _ = _sc_kernel(xT, *tvs)
    return jnp.transpose(outk, (1, 3, 0, 2)).reshape(_N, _N_LEVELS * _N_FEATS)
